# baseline (speedup 1.0000x reference)
import functools

import jax
import jax.numpy as jnp
from jax import lax
from jax.experimental import pallas as pl
from jax.experimental.pallas import tpu as pltpu

_D = 768
_DFF = 1536
_E = 8
_NTOK = 2048
_K = 2
_S = _NTOK * _K
_BT = 256
_BG = 256
_NB = _S // _BG + _E
_PMAX = _NB * _BG


def _gating_body(x_ref, gw_ref, gb_ref, wts_ref, idx_ref):
    x = x_ref[...]
    logits = jnp.dot(x, gw_ref[...],
                     preferred_element_type=jnp.float32) + gb_ref[...]
    iota = lax.broadcasted_iota(jnp.int32, (_BT, _E), 1)
    m1 = jnp.max(logits, axis=1, keepdims=True)
    i1 = jnp.min(jnp.where(logits >= m1, iota, _E), axis=1, keepdims=True)
    l2 = jnp.where(iota == i1, -jnp.inf, logits)
    m2 = jnp.max(l2, axis=1, keepdims=True)
    i2 = jnp.min(jnp.where(l2 >= m2, iota, _E), axis=1, keepdims=True)
    z = jnp.sum(jnp.exp(logits - m1), axis=1, keepdims=True)
    p1 = 1.0 / z
    p2 = jnp.exp(m2 - m1) / z
    t = jnp.exp(p2 - p1)
    w1 = 1.0 / (1.0 + t)
    w2 = t / (1.0 + t)
    wts_ref[...] = jnp.concatenate([w1, w2], axis=1)
    idx_ref[...] = jnp.concatenate([i1, i2], axis=1)


def _gating(x, gate_w, gate_b):
    return pl.pallas_call(
        _gating_body,
        grid=(_NTOK // _BT,),
        in_specs=[
            pl.BlockSpec((_BT, _D), lambda n: (n, 0)),
            pl.BlockSpec((_D, _E), lambda n: (0, 0)),
            pl.BlockSpec((1, _E), lambda n: (0, 0)),
        ],
        out_specs=[
            pl.BlockSpec((_BT, _K), lambda n: (n, 0)),
            pl.BlockSpec((_BT, _K), lambda n: (n, 0)),
        ],
        out_shape=[
            jax.ShapeDtypeStruct((_NTOK, _K), jnp.float32),
            jax.ShapeDtypeStruct((_NTOK, _K), jnp.int32),
        ],
        compiler_params=pltpu.CompilerParams(
            dimension_semantics=("parallel",),
        ),
    )(x, gate_w, gate_b.reshape(1, _E))


def _routing(idx, wts):
    eflat = idx.reshape(_S)
    wflat = wts.reshape(_S)
    oh = (eflat[:, None] == jnp.arange(_E)[None, :]).astype(jnp.int32)
    counts = jnp.sum(oh, axis=0)
    padded = ((counts + _BG - 1) // _BG) * _BG
    starts = jnp.concatenate([jnp.zeros((1,), jnp.int32),
                              jnp.cumsum(padded)[:-1].astype(jnp.int32)])
    rank = jnp.sum(jnp.where(oh == 1, jnp.cumsum(oh, axis=0) - 1, 0), axis=1)
    pos_flat = starts[eflat] + rank
    tok = jnp.arange(_S, dtype=jnp.int32) // _K
    gather_tok = jnp.zeros((_PMAX,), jnp.int32).at[pos_flat].set(tok)
    wslot = jnp.zeros((_PMAX,), jnp.float32).at[pos_flat].set(wflat)
    ends = starts + padded
    bstart = jnp.arange(_NB, dtype=jnp.int32) * _BG
    block_expert = jnp.sum((bstart[:, None] >= ends[None, :]).astype(jnp.int32),
                           axis=1)
    block_expert = jnp.minimum(block_expert, _E - 1)
    chg = jnp.concatenate([jnp.ones((1,), jnp.int32),
                           (block_expert[1:] != block_expert[:-1]).astype(jnp.int32)])
    bix = (jnp.cumsum(chg) - 1) % 2
    return gather_tok, wslot, block_expert, chg, bix.astype(jnp.int32), \
        pos_flat.reshape(_NTOK, _K)


def _ffn_body(be_ref, chg_ref, bix_ref, x_ref, w_ref, b1_ref, b2_ref,
              W1_hbm, W2_hbm, y_ref, w1buf, w2buf, sems):
    b = pl.program_id(0)

    @pl.when(b == 0)
    def _():
        pltpu.make_async_copy(W1_hbm.at[be_ref[0]], w1buf.at[0], sems.at[0]).start()
        pltpu.make_async_copy(W2_hbm.at[be_ref[0]], w2buf.at[0], sems.at[0]).start()

    @pl.when(jnp.logical_and(b + 1 < _NB, chg_ref[jnp.minimum(b + 1, _NB - 1)] == 1))
    def _():
        nxt = jnp.minimum(b + 1, _NB - 1)
        pltpu.make_async_copy(W1_hbm.at[be_ref[nxt]], w1buf.at[bix_ref[nxt]],
                              sems.at[bix_ref[nxt]]).start()
        pltpu.make_async_copy(W2_hbm.at[be_ref[nxt]], w2buf.at[bix_ref[nxt]],
                              sems.at[bix_ref[nxt]]).start()

    @pl.when(chg_ref[b] == 1)
    def _():
        pltpu.make_async_copy(W1_hbm.at[be_ref[b]], w1buf.at[bix_ref[b]],
                              sems.at[bix_ref[b]]).wait()
        pltpu.make_async_copy(W2_hbm.at[be_ref[b]], w2buf.at[bix_ref[b]],
                              sems.at[bix_ref[b]]).wait()

    bix = bix_ref[b]
    x = x_ref[...].astype(jnp.bfloat16)
    h = jnp.maximum(jnp.dot(x, w1buf[bix].astype(jnp.bfloat16),
                            preferred_element_type=jnp.float32) + b1_ref[0], 0.0)
    y = jnp.dot(h.astype(jnp.bfloat16), w2buf[bix].astype(jnp.bfloat16),
                preferred_element_type=jnp.float32) + b2_ref[0]
    y_ref[...] = y * w_ref[...]


def _ffn(x_g, wslot, block_expert, chg, bix, W1, b1, W2, b2):
    grid_spec = pltpu.PrefetchScalarGridSpec(
        num_scalar_prefetch=3,
        grid=(_NB,),
        in_specs=[
            pl.BlockSpec((_BG, _D), lambda b, be, chg, bix: (b, 0)),
            pl.BlockSpec((_BG, 1), lambda b, be, chg, bix: (b, 0)),
            pl.BlockSpec((1, 1, _DFF), lambda b, be, chg, bix: (be[b], 0, 0)),
            pl.BlockSpec((1, 1, _D), lambda b, be, chg, bix: (be[b], 0, 0)),
            pl.BlockSpec(memory_space=pl.ANY),
            pl.BlockSpec(memory_space=pl.ANY),
        ],
        out_specs=pl.BlockSpec((_BG, _D), lambda b, be, chg, bix: (b, 0)),
        scratch_shapes=[
            pltpu.VMEM((2, _D, _DFF), jnp.float32),
            pltpu.VMEM((2, _DFF, _D), jnp.float32),
            pltpu.SemaphoreType.DMA((2,)),
        ],
    )
    return pl.pallas_call(
        _ffn_body,
        grid_spec=grid_spec,
        out_shape=jax.ShapeDtypeStruct((_PMAX, _D), jnp.float32),
        compiler_params=pltpu.CompilerParams(
            dimension_semantics=("arbitrary",),
        ),
    )(block_expert, chg, bix, x_g, wslot.reshape(_PMAX, 1),
      b1.reshape(_E, 1, _DFF), b2.reshape(_E, 1, _D), W1, W2)


def kernel(x, gate_w, gate_b, W1, b1, W2, b2):
    wts, idx = _gating(x, gate_w, gate_b)
    gather_tok, wslot, block_expert, chg, bix, pos = _routing(idx, wts)
    x_g = jnp.take(x, gather_tok, axis=0, mode="clip")
    y_w = _ffn(x_g, wslot, block_expert, chg, bix, W1, b1, W2, b2)
    out = (jnp.take(y_w, pos[:, 0], axis=0, mode="clip")
           + jnp.take(y_w, pos[:, 1], axis=0, mode="clip"))
    return out


# speedup vs baseline: 1.2039x; 1.2039x over previous
import functools

import jax
import jax.numpy as jnp
from jax import lax
from jax.experimental import pallas as pl
from jax.experimental.pallas import tpu as pltpu

_D = 768
_DFF = 1536
_E = 8
_NTOK = 2048
_K = 2
_S = _NTOK * _K
_BT = 256
_BG = 256
_NB = _S // _BG + _E
_PMAX = _NB * _BG


def _gating_body(x_ref, gw_ref, gb_ref, wts_ref, idx_ref):
    x = x_ref[...]
    logits = jnp.dot(x, gw_ref[...],
                     preferred_element_type=jnp.float32) + gb_ref[...]
    iota = lax.broadcasted_iota(jnp.int32, (_BT, _E), 1)
    m1 = jnp.max(logits, axis=1, keepdims=True)
    i1 = jnp.min(jnp.where(logits >= m1, iota, _E), axis=1, keepdims=True)
    l2 = jnp.where(iota == i1, -jnp.inf, logits)
    m2 = jnp.max(l2, axis=1, keepdims=True)
    i2 = jnp.min(jnp.where(l2 >= m2, iota, _E), axis=1, keepdims=True)
    z = jnp.sum(jnp.exp(logits - m1), axis=1, keepdims=True)
    p1 = 1.0 / z
    p2 = jnp.exp(m2 - m1) / z
    t = jnp.exp(p2 - p1)
    w1 = 1.0 / (1.0 + t)
    w2 = t / (1.0 + t)
    wts_ref[...] = jnp.concatenate([w1, w2], axis=1)
    idx_ref[...] = jnp.concatenate([i1, i2], axis=1)


def _gating(x, gate_w, gate_b):
    return pl.pallas_call(
        _gating_body,
        grid=(_NTOK // _BT,),
        in_specs=[
            pl.BlockSpec((_BT, _D), lambda n: (n, 0)),
            pl.BlockSpec((_D, _E), lambda n: (0, 0)),
            pl.BlockSpec((1, _E), lambda n: (0, 0)),
        ],
        out_specs=[
            pl.BlockSpec((_BT, _K), lambda n: (n, 0)),
            pl.BlockSpec((_BT, _K), lambda n: (n, 0)),
        ],
        out_shape=[
            jax.ShapeDtypeStruct((_NTOK, _K), jnp.float32),
            jax.ShapeDtypeStruct((_NTOK, _K), jnp.int32),
        ],
        compiler_params=pltpu.CompilerParams(
            dimension_semantics=("parallel",),
        ),
    )(x, gate_w, gate_b.reshape(1, _E))


def _plan_body(idx_ref, pos_ref, binfo_ref):
    idx = idx_ref[...]
    i1 = idx[:, 0:1]
    i2 = idx[:, 1:2]
    lane8 = lax.broadcasted_iota(jnp.int32, (_NTOK, _E), 1)
    oh1 = (i1 == lane8).astype(jnp.float32)
    oh2 = (i2 == lane8).astype(jnp.float32)
    ohs = oh1 + oh2
    C = 256
    r_io = lax.broadcasted_iota(jnp.int32, (C, C), 0)
    c_io = lax.broadcasted_iota(jnp.int32, (C, C), 1)
    T = (c_io < r_io).astype(jnp.float32)
    carry = jnp.zeros((1, _E), jnp.float32)
    chunks = []
    for i in range(_NTOK // C):
        chunk = ohs[i * C:(i + 1) * C]
        chunks.append(jnp.dot(T, chunk, preferred_element_type=jnp.float32) + carry)
        carry = carry + jnp.sum(chunk, axis=0, keepdims=True)
    rank_base = jnp.concatenate(chunks, axis=0)
    counts = carry
    padded = jnp.floor((counts + (_BG - 1)) / _BG) * _BG
    r8 = lax.broadcasted_iota(jnp.int32, (_E, _E), 0)
    c8 = lax.broadcasted_iota(jnp.int32, (_E, _E), 1)
    U8 = (r8 < c8).astype(jnp.float32)
    starts = jnp.dot(padded, U8, preferred_element_type=jnp.float32)
    pos1 = jnp.sum(oh1 * (rank_base + starts), axis=1, keepdims=True)
    pos2 = jnp.sum(oh2 * (rank_base + starts), axis=1, keepdims=True)
    pos_ref[...] = jnp.concatenate([pos1, pos2], axis=1).astype(jnp.int32)
    rbi = lax.broadcasted_iota(jnp.int32, (_NB, _E), 0)
    bstart = (rbi * _BG).astype(jnp.float32)
    ends = starts + padded
    be = jnp.minimum(jnp.sum((bstart >= ends).astype(jnp.int32), axis=1,
                             keepdims=True), _E - 1)
    total = jnp.sum(padded)
    valid = (bstart[:, 0:1] < total).astype(jnp.int32)
    chg = jnp.concatenate(
        [jnp.ones((1, 1), jnp.int32), (be[1:] != be[:-1]).astype(jnp.int32)],
        axis=0)
    r24 = lax.broadcasted_iota(jnp.int32, (_NB, _NB), 0)
    c24 = lax.broadcasted_iota(jnp.int32, (_NB, _NB), 1)
    Tinc = (c24 <= r24).astype(jnp.float32)
    csum = jnp.dot(Tinc, chg.astype(jnp.float32), preferred_element_type=jnp.float32)
    bix = (csum.astype(jnp.int32) - 1) % 2
    binfo_ref[...] = jnp.concatenate(
        [be, chg, bix, valid, jnp.zeros((_NB, _E - 4), jnp.int32)], axis=1)


def _plan(idx):
    return pl.pallas_call(
        _plan_body,
        grid=(1,),
        in_specs=[pl.BlockSpec((_NTOK, _K), lambda i: (0, 0))],
        out_specs=[
            pl.BlockSpec((_NTOK, _K), lambda i: (0, 0)),
            pl.BlockSpec((_NB, _E), lambda i: (0, 0)),
        ],
        out_shape=[
            jax.ShapeDtypeStruct((_NTOK, _K), jnp.int32),
            jax.ShapeDtypeStruct((_NB, _E), jnp.int32),
        ],
    )(idx)


def _ffn_body(be_ref, chg_ref, bix_ref, x_ref, b1_ref, b2_ref,
              W1_hbm, W2_hbm, y_ref, w1buf, w2buf, sems):
    b = pl.program_id(0)

    @pl.when(b == 0)
    def _():
        pltpu.make_async_copy(W1_hbm.at[be_ref[0]], w1buf.at[0], sems.at[0]).start()
        pltpu.make_async_copy(W2_hbm.at[be_ref[0]], w2buf.at[0], sems.at[0]).start()

    @pl.when(jnp.logical_and(b + 1 < _NB, chg_ref[jnp.minimum(b + 1, _NB - 1)] == 1))
    def _():
        nxt = jnp.minimum(b + 1, _NB - 1)
        pltpu.make_async_copy(W1_hbm.at[be_ref[nxt]], w1buf.at[bix_ref[nxt]],
                              sems.at[bix_ref[nxt]]).start()
        pltpu.make_async_copy(W2_hbm.at[be_ref[nxt]], w2buf.at[bix_ref[nxt]],
                              sems.at[bix_ref[nxt]]).start()

    @pl.when(chg_ref[b] == 1)
    def _():
        pltpu.make_async_copy(W1_hbm.at[be_ref[b]], w1buf.at[bix_ref[b]],
                              sems.at[bix_ref[b]]).wait()
        pltpu.make_async_copy(W2_hbm.at[be_ref[b]], w2buf.at[bix_ref[b]],
                              sems.at[bix_ref[b]]).wait()

    bix = bix_ref[b]
    x = x_ref[...].astype(jnp.bfloat16)
    h = jnp.maximum(jnp.dot(x, w1buf[bix].astype(jnp.bfloat16),
                            preferred_element_type=jnp.float32) + b1_ref[0], 0.0)
    y = jnp.dot(h.astype(jnp.bfloat16), w2buf[bix].astype(jnp.bfloat16),
                preferred_element_type=jnp.float32) + b2_ref[0]
    y_ref[...] = y


def _ffn(x_g, block_expert, chg, bix, W1, b1, W2, b2):
    grid_spec = pltpu.PrefetchScalarGridSpec(
        num_scalar_prefetch=3,
        grid=(_NB,),
        in_specs=[
            pl.BlockSpec((_BG, _D), lambda b, be, chg, bix: (b, 0)),
            pl.BlockSpec((1, 1, _DFF), lambda b, be, chg, bix: (be[b], 0, 0)),
            pl.BlockSpec((1, 1, _D), lambda b, be, chg, bix: (be[b], 0, 0)),
            pl.BlockSpec(memory_space=pl.ANY),
            pl.BlockSpec(memory_space=pl.ANY),
        ],
        out_specs=pl.BlockSpec((_BG, _D), lambda b, be, chg, bix: (b, 0)),
        scratch_shapes=[
            pltpu.VMEM((2, _D, _DFF), jnp.float32),
            pltpu.VMEM((2, _DFF, _D), jnp.float32),
            pltpu.SemaphoreType.DMA((2,)),
        ],
    )
    return pl.pallas_call(
        _ffn_body,
        grid_spec=grid_spec,
        out_shape=jax.ShapeDtypeStruct((_PMAX, _D), jnp.float32),
        compiler_params=pltpu.CompilerParams(
            dimension_semantics=("arbitrary",),
        ),
    )(block_expert, chg, bix, x_g,
      b1.reshape(_E, 1, _DFF), b2.reshape(_E, 1, _D), W1, W2)


def kernel(x, gate_w, gate_b, W1, b1, W2, b2):
    wts, idx = _gating(x, gate_w, gate_b)
    pos, binfo = _plan(idx)
    block_expert = binfo[:, 0]
    chg = binfo[:, 1]
    bix = binfo[:, 2]
    x_g = (jnp.zeros((_PMAX, _D), jnp.float32)
           .at[pos[:, 0]].set(x).at[pos[:, 1]].set(x))
    y = _ffn(x_g, block_expert, chg, bix, W1, b1, W2, b2)
    out = (wts[:, 0:1] * jnp.take(y, pos[:, 0], axis=0, mode="clip")
           + wts[:, 1:2] * jnp.take(y, pos[:, 1], axis=0, mode="clip"))
    return out


# speedup vs baseline: 1.2917x; 1.0729x over previous
import functools

import jax
import jax.numpy as jnp
from jax import lax
from jax.experimental import pallas as pl
from jax.experimental.pallas import tpu as pltpu
from jax.experimental.pallas import tpu_sc as plsc

_D = 768
_DFF = 1536
_E = 8
_NTOK = 2048
_K = 2
_S = _NTOK * _K
_BT = 256
_BG = 256
_NB = _S // _BG + _E
_PMAX = _NB * _BG


def _gating_body(x_ref, gw_ref, gb_ref, wts_ref, idx_ref):
    x = x_ref[...]
    logits = jnp.dot(x, gw_ref[...],
                     preferred_element_type=jnp.float32) + gb_ref[...]
    iota = lax.broadcasted_iota(jnp.int32, (_BT, _E), 1)
    m1 = jnp.max(logits, axis=1, keepdims=True)
    i1 = jnp.min(jnp.where(logits >= m1, iota, _E), axis=1, keepdims=True)
    l2 = jnp.where(iota == i1, -jnp.inf, logits)
    m2 = jnp.max(l2, axis=1, keepdims=True)
    i2 = jnp.min(jnp.where(l2 >= m2, iota, _E), axis=1, keepdims=True)
    z = jnp.sum(jnp.exp(logits - m1), axis=1, keepdims=True)
    p1 = 1.0 / z
    p2 = jnp.exp(m2 - m1) / z
    t = jnp.exp(p2 - p1)
    w1 = 1.0 / (1.0 + t)
    w2 = t / (1.0 + t)
    wts_ref[...] = jnp.concatenate([w1, w2], axis=1)
    idx_ref[...] = jnp.concatenate([i1, i2], axis=1)


def _gating(x, gate_w, gate_b):
    return pl.pallas_call(
        _gating_body,
        grid=(_NTOK // _BT,),
        in_specs=[
            pl.BlockSpec((_BT, _D), lambda n: (n, 0)),
            pl.BlockSpec((_D, _E), lambda n: (0, 0)),
            pl.BlockSpec((1, _E), lambda n: (0, 0)),
        ],
        out_specs=[
            pl.BlockSpec((_BT, _K), lambda n: (n, 0)),
            pl.BlockSpec((_BT, _K), lambda n: (n, 0)),
        ],
        out_shape=[
            jax.ShapeDtypeStruct((_NTOK, _K), jnp.float32),
            jax.ShapeDtypeStruct((_NTOK, _K), jnp.int32),
        ],
        compiler_params=pltpu.CompilerParams(
            dimension_semantics=("parallel",),
        ),
    )(x, gate_w, gate_b.reshape(1, _E))


def _plan_body(idx_ref, pos_ref, binfo_ref):
    idx = idx_ref[...]
    i1 = idx[:, 0:1]
    i2 = idx[:, 1:2]
    lane8 = lax.broadcasted_iota(jnp.int32, (_NTOK, _E), 1)
    oh1 = (i1 == lane8).astype(jnp.float32)
    oh2 = (i2 == lane8).astype(jnp.float32)
    ohs = oh1 + oh2
    C = 256
    r_io = lax.broadcasted_iota(jnp.int32, (C, C), 0)
    c_io = lax.broadcasted_iota(jnp.int32, (C, C), 1)
    T = (c_io < r_io).astype(jnp.float32)
    carry = jnp.zeros((1, _E), jnp.float32)
    chunks = []
    for i in range(_NTOK // C):
        chunk = ohs[i * C:(i + 1) * C]
        chunks.append(jnp.dot(T, chunk, preferred_element_type=jnp.float32) + carry)
        carry = carry + jnp.sum(chunk, axis=0, keepdims=True)
    rank_base = jnp.concatenate(chunks, axis=0)
    counts = carry
    padded = jnp.floor((counts + (_BG - 1)) / _BG) * _BG
    r8 = lax.broadcasted_iota(jnp.int32, (_E, _E), 0)
    c8 = lax.broadcasted_iota(jnp.int32, (_E, _E), 1)
    U8 = (r8 < c8).astype(jnp.float32)
    starts = jnp.dot(padded, U8, preferred_element_type=jnp.float32)
    pos1 = jnp.sum(oh1 * (rank_base + starts), axis=1, keepdims=True)
    pos2 = jnp.sum(oh2 * (rank_base + starts), axis=1, keepdims=True)
    pos_ref[...] = jnp.concatenate([pos1, pos2], axis=1).astype(jnp.int32)
    rbi = lax.broadcasted_iota(jnp.int32, (_NB, _E), 0)
    bstart = (rbi * _BG).astype(jnp.float32)
    ends = starts + padded
    be = jnp.minimum(jnp.sum((bstart >= ends).astype(jnp.int32), axis=1,
                             keepdims=True), _E - 1)
    total = jnp.sum(padded)
    valid = (bstart[:, 0:1] < total).astype(jnp.int32)
    chg = jnp.concatenate(
        [jnp.ones((1, 1), jnp.int32), (be[1:] != be[:-1]).astype(jnp.int32)],
        axis=0)
    r24 = lax.broadcasted_iota(jnp.int32, (_NB, _NB), 0)
    c24 = lax.broadcasted_iota(jnp.int32, (_NB, _NB), 1)
    Tinc = (c24 <= r24).astype(jnp.float32)
    csum = jnp.dot(Tinc, chg.astype(jnp.float32), preferred_element_type=jnp.float32)
    bix = (csum.astype(jnp.int32) - 1) % 2
    binfo_ref[...] = jnp.concatenate(
        [be, chg, bix, valid, jnp.zeros((_NB, _E - 4), jnp.int32)], axis=1)


def _plan(idx):
    return pl.pallas_call(
        _plan_body,
        grid=(1,),
        in_specs=[pl.BlockSpec((_NTOK, _K), lambda i: (0, 0))],
        out_specs=[
            pl.BlockSpec((_NTOK, _K), lambda i: (0, 0)),
            pl.BlockSpec((_NB, _E), lambda i: (0, 0)),
        ],
        out_shape=[
            jax.ShapeDtypeStruct((_NTOK, _K), jnp.int32),
            jax.ShapeDtypeStruct((_NB, _E), jnp.int32),
        ],
    )(idx)


def _ffn_body(be_ref, chg_ref, bix_ref, x_ref, w_ref, b1_ref, b2_ref,
              W1_hbm, W2_hbm, y_ref, w1buf, w2buf, sems):
    b = pl.program_id(0)

    @pl.when(b == 0)
    def _():
        pltpu.make_async_copy(W1_hbm.at[be_ref[0]], w1buf.at[0], sems.at[0]).start()
        pltpu.make_async_copy(W2_hbm.at[be_ref[0]], w2buf.at[0], sems.at[0]).start()

    @pl.when(jnp.logical_and(b + 1 < _NB, chg_ref[jnp.minimum(b + 1, _NB - 1)] == 1))
    def _():
        nxt = jnp.minimum(b + 1, _NB - 1)
        pltpu.make_async_copy(W1_hbm.at[be_ref[nxt]], w1buf.at[bix_ref[nxt]],
                              sems.at[bix_ref[nxt]]).start()
        pltpu.make_async_copy(W2_hbm.at[be_ref[nxt]], w2buf.at[bix_ref[nxt]],
                              sems.at[bix_ref[nxt]]).start()

    @pl.when(chg_ref[b] == 1)
    def _():
        pltpu.make_async_copy(W1_hbm.at[be_ref[b]], w1buf.at[bix_ref[b]],
                              sems.at[bix_ref[b]]).wait()
        pltpu.make_async_copy(W2_hbm.at[be_ref[b]], w2buf.at[bix_ref[b]],
                              sems.at[bix_ref[b]]).wait()

    bix = bix_ref[b]
    x = x_ref[...].astype(jnp.bfloat16)
    h = jnp.maximum(jnp.dot(x, w1buf[bix].astype(jnp.bfloat16),
                            preferred_element_type=jnp.float32) + b1_ref[0], 0.0)
    y = jnp.dot(h.astype(jnp.bfloat16), w2buf[bix].astype(jnp.bfloat16),
                preferred_element_type=jnp.float32) + b2_ref[0]
    y_ref[...] = y * w_ref[...]


def _ffn(x_g, wslot, block_expert, chg, bix, W1, b1, W2, b2):
    grid_spec = pltpu.PrefetchScalarGridSpec(
        num_scalar_prefetch=3,
        grid=(_NB,),
        in_specs=[
            pl.BlockSpec((_BG, _D), lambda b, be, chg, bix: (b, 0)),
            pl.BlockSpec((_BG, 1), lambda b, be, chg, bix: (b, 0)),
            pl.BlockSpec((1, 1, _DFF), lambda b, be, chg, bix: (be[b], 0, 0)),
            pl.BlockSpec((1, 1, _D), lambda b, be, chg, bix: (be[b], 0, 0)),
            pl.BlockSpec(memory_space=pl.ANY),
            pl.BlockSpec(memory_space=pl.ANY),
        ],
        out_specs=pl.BlockSpec((_BG, _D), lambda b, be, chg, bix: (b, 0)),
        scratch_shapes=[
            pltpu.VMEM((2, _D, _DFF), jnp.float32),
            pltpu.VMEM((2, _DFF, _D), jnp.float32),
            pltpu.SemaphoreType.DMA((2,)),
        ],
    )
    return pl.pallas_call(
        _ffn_body,
        grid_spec=grid_spec,
        out_shape=jax.ShapeDtypeStruct((_PMAX, _D), jnp.float32),
        compiler_params=pltpu.CompilerParams(
            dimension_semantics=("arbitrary",),
        ),
    )(block_expert, chg, bix, x_g, wslot.reshape(_PMAX, 1),
      b1.reshape(_E, 1, _DFF), b2.reshape(_E, 1, _D), W1, W2)


_NW = 32
_TPW = _NTOK // _NW
_L = 16
_MESH = plsc.VectorSubcoreMesh(core_axis_name="c", subcore_axis_name="s")


def _wid():
    return lax.axis_index("s") * 2 + lax.axis_index("c")


@functools.partial(
    pl.kernel, mesh=_MESH,
    out_type=[
        jax.ShapeDtypeStruct((_PMAX, _D), jnp.float32),
        jax.ShapeDtypeStruct((_PMAX,), jnp.float32),
    ],
    scratch_types=[
        pltpu.VMEM((_TPW,), jnp.int32),
        pltpu.VMEM((_TPW,), jnp.int32),
        pltpu.VMEM((_TPW,), jnp.float32),
        pltpu.VMEM((_TPW,), jnp.float32),
        pltpu.VMEM((_TPW, _D), jnp.float32),
        pltpu.SemaphoreType.DMA,
        pltpu.SemaphoreType.DMA,
        pltpu.SemaphoreType.DMA,
    ],
)
def _sc_dispatch(x_hbm, pos0_hbm, pos1_hbm, w0_hbm, w1_hbm,
                 xg_hbm, wslot_hbm,
                 p0_v, p1_v, w0_v, w1_v, rows_v, sem0, sem1, semw):
    base = _wid() * _TPW
    pltpu.sync_copy(pos0_hbm.at[pl.ds(base, _TPW)], p0_v)
    pltpu.sync_copy(pos1_hbm.at[pl.ds(base, _TPW)], p1_v)
    pltpu.sync_copy(w0_hbm.at[pl.ds(base, _TPW)], w0_v)
    pltpu.sync_copy(w1_hbm.at[pl.ds(base, _TPW)], w1_v)
    pltpu.sync_copy(x_hbm.at[pl.ds(base, _TPW)], rows_v)
    c0 = pltpu.async_copy(rows_v, xg_hbm.at[p0_v], sem0)
    c1 = pltpu.async_copy(rows_v, xg_hbm.at[p1_v], sem1)
    cw0 = pltpu.async_copy(w0_v, wslot_hbm.at[p0_v], semw)
    cw1 = pltpu.async_copy(w1_v, wslot_hbm.at[p1_v], semw)
    c0.wait()
    c1.wait()
    cw0.wait()
    cw1.wait()


@functools.partial(
    pl.kernel, mesh=_MESH,
    out_type=jax.ShapeDtypeStruct((_NTOK, _D), jnp.float32),
    scratch_types=[
        pltpu.VMEM((_TPW,), jnp.int32),
        pltpu.VMEM((_TPW,), jnp.int32),
        pltpu.VMEM((_TPW, _D), jnp.float32),
        pltpu.VMEM((_TPW, _D), jnp.float32),
        pltpu.SemaphoreType.DMA,
        pltpu.SemaphoreType.DMA,
    ],
)
def _sc_combine(y_hbm, pos0_hbm, pos1_hbm, out_hbm,
                p0_v, p1_v, y0_v, y1_v, sem0, sem1):
    base = _wid() * _TPW
    pltpu.sync_copy(pos0_hbm.at[pl.ds(base, _TPW)], p0_v)
    pltpu.sync_copy(pos1_hbm.at[pl.ds(base, _TPW)], p1_v)
    c0 = pltpu.async_copy(y_hbm.at[p0_v], y0_v, sem0)
    c1 = pltpu.async_copy(y_hbm.at[p1_v], y1_v, sem1)
    c0.wait()
    c1.wait()

    def row(r, _):
        for c in range(_D // _L):
            sl = pl.ds(c * _L, _L)
            y0_v[r, sl] = y0_v[r, sl] + y1_v[r, sl]
        return 0

    lax.fori_loop(0, _TPW, row, 0)
    pltpu.sync_copy(y0_v, out_hbm.at[pl.ds(base, _TPW)])


def kernel(x, gate_w, gate_b, W1, b1, W2, b2):
    wts, idx = _gating(x, gate_w, gate_b)
    pos, binfo = _plan(idx)
    block_expert = binfo[:, 0]
    chg = binfo[:, 1]
    bix = binfo[:, 2]
    p0 = pos[:, 0]
    p1 = pos[:, 1]
    x_g, wslot = _sc_dispatch(x, p0, p1, wts[:, 0], wts[:, 1])
    y = _ffn(x_g, wslot, block_expert, chg, bix, W1, b1, W2, b2)
    return _sc_combine(y, p0, p1)


# speedup vs baseline: 1.3274x; 1.0276x over previous
import functools

import jax
import jax.numpy as jnp
from jax import lax
from jax.experimental import pallas as pl
from jax.experimental.pallas import tpu as pltpu
from jax.experimental.pallas import tpu_sc as plsc

_D = 768
_DFF = 1536
_E = 8
_NTOK = 2048
_K = 2
_S = _NTOK * _K
_BT = 256
_BG = 256
_NB = _S // _BG + _E
_PMAX = _NB * _BG


def _gating_body(x_ref, gw_ref, gb_ref, wts_ref, idx_ref):
    x = x_ref[...]
    logits = jnp.dot(x, gw_ref[...],
                     preferred_element_type=jnp.float32) + gb_ref[...]
    iota = lax.broadcasted_iota(jnp.int32, (_BT, _E), 1)
    m1 = jnp.max(logits, axis=1, keepdims=True)
    i1 = jnp.min(jnp.where(logits >= m1, iota, _E), axis=1, keepdims=True)
    l2 = jnp.where(iota == i1, -jnp.inf, logits)
    m2 = jnp.max(l2, axis=1, keepdims=True)
    i2 = jnp.min(jnp.where(l2 >= m2, iota, _E), axis=1, keepdims=True)
    z = jnp.sum(jnp.exp(logits - m1), axis=1, keepdims=True)
    p1 = 1.0 / z
    p2 = jnp.exp(m2 - m1) / z
    t = jnp.exp(p2 - p1)
    w1 = 1.0 / (1.0 + t)
    w2 = t / (1.0 + t)
    wts_ref[...] = jnp.concatenate([w1, w2], axis=1)
    idx_ref[...] = jnp.concatenate([i1, i2], axis=1)


def _gating(x, gate_w, gate_b):
    return pl.pallas_call(
        _gating_body,
        grid=(_NTOK // _BT,),
        in_specs=[
            pl.BlockSpec((_BT, _D), lambda n: (n, 0)),
            pl.BlockSpec((_D, _E), lambda n: (0, 0)),
            pl.BlockSpec((1, _E), lambda n: (0, 0)),
        ],
        out_specs=[
            pl.BlockSpec((_BT, _K), lambda n: (n, 0)),
            pl.BlockSpec((_BT, _K), lambda n: (n, 0)),
        ],
        out_shape=[
            jax.ShapeDtypeStruct((_NTOK, _K), jnp.float32),
            jax.ShapeDtypeStruct((_NTOK, _K), jnp.int32),
        ],
        compiler_params=pltpu.CompilerParams(
            dimension_semantics=("parallel",),
        ),
    )(x, gate_w, gate_b.reshape(1, _E))


def _plan_body(idx_ref, pos_ref, binfo_ref):
    idx = idx_ref[...]
    i1 = idx[:, 0:1]
    i2 = idx[:, 1:2]
    lane8 = lax.broadcasted_iota(jnp.int32, (_NTOK, _E), 1)
    oh1 = (i1 == lane8).astype(jnp.float32)
    oh2 = (i2 == lane8).astype(jnp.float32)
    ohs = oh1 + oh2
    C = 256
    r_io = lax.broadcasted_iota(jnp.int32, (C, C), 0)
    c_io = lax.broadcasted_iota(jnp.int32, (C, C), 1)
    T = (c_io < r_io).astype(jnp.float32)
    carry = jnp.zeros((1, _E), jnp.float32)
    chunks = []
    for i in range(_NTOK // C):
        chunk = ohs[i * C:(i + 1) * C]
        chunks.append(jnp.dot(T, chunk, preferred_element_type=jnp.float32) + carry)
        carry = carry + jnp.sum(chunk, axis=0, keepdims=True)
    rank_base = jnp.concatenate(chunks, axis=0)
    counts = carry
    padded = jnp.floor((counts + (_BG - 1)) / _BG) * _BG
    r8 = lax.broadcasted_iota(jnp.int32, (_E, _E), 0)
    c8 = lax.broadcasted_iota(jnp.int32, (_E, _E), 1)
    U8 = (r8 < c8).astype(jnp.float32)
    starts = jnp.dot(padded, U8, preferred_element_type=jnp.float32)
    pos1 = jnp.sum(oh1 * (rank_base + starts), axis=1, keepdims=True)
    pos2 = jnp.sum(oh2 * (rank_base + starts), axis=1, keepdims=True)
    pos_ref[...] = jnp.concatenate([pos1, pos2], axis=1).astype(jnp.int32)
    rbi = lax.broadcasted_iota(jnp.int32, (_NB, _E), 0)
    bstart = (rbi * _BG).astype(jnp.float32)
    ends = starts + padded
    be = jnp.minimum(jnp.sum((bstart >= ends).astype(jnp.int32), axis=1,
                             keepdims=True), _E - 1)
    total = jnp.sum(padded)
    valid = (bstart[:, 0:1] < total).astype(jnp.int32)
    chg = jnp.concatenate(
        [jnp.ones((1, 1), jnp.int32), (be[1:] != be[:-1]).astype(jnp.int32)],
        axis=0)
    r24 = lax.broadcasted_iota(jnp.int32, (_NB, _NB), 0)
    c24 = lax.broadcasted_iota(jnp.int32, (_NB, _NB), 1)
    Tinc = (c24 <= r24).astype(jnp.float32)
    csum = jnp.dot(Tinc, chg.astype(jnp.float32), preferred_element_type=jnp.float32)
    bix = (csum.astype(jnp.int32) - 1) % 2
    binfo_ref[...] = jnp.concatenate(
        [be, chg, bix, valid, jnp.zeros((_NB, _E - 4), jnp.int32)], axis=1)


def _plan(idx):
    return pl.pallas_call(
        _plan_body,
        grid=(1,),
        in_specs=[pl.BlockSpec((_NTOK, _K), lambda i: (0, 0))],
        out_specs=[
            pl.BlockSpec((_NTOK, _K), lambda i: (0, 0)),
            pl.BlockSpec((_NB, _E), lambda i: (0, 0)),
        ],
        out_shape=[
            jax.ShapeDtypeStruct((_NTOK, _K), jnp.int32),
            jax.ShapeDtypeStruct((_NB, _E), jnp.int32),
        ],
    )(idx)


def _ffn_body(be_ref, chg_ref, bix_ref, vld_ref, x_ref, w_ref, b1_ref, b2_ref,
              W1_hbm, W2_hbm, y_ref, w1buf, w2buf, sems):
    b = pl.program_id(0)

    @pl.when(b == 0)
    def _():
        pltpu.make_async_copy(W1_hbm.at[be_ref[0]], w1buf.at[0], sems.at[0]).start()
        pltpu.make_async_copy(W2_hbm.at[be_ref[0]], w2buf.at[0], sems.at[0]).start()

    nxt = jnp.minimum(b + 1, _NB - 1)

    @pl.when(jnp.logical_and(b + 1 < _NB,
                             (chg_ref[nxt] == 1) & (vld_ref[nxt] == 1)))
    def _():
        pltpu.make_async_copy(W1_hbm.at[be_ref[nxt]], w1buf.at[bix_ref[nxt]],
                              sems.at[bix_ref[nxt]]).start()
        pltpu.make_async_copy(W2_hbm.at[be_ref[nxt]], w2buf.at[bix_ref[nxt]],
                              sems.at[bix_ref[nxt]]).start()

    @pl.when((chg_ref[b] == 1) & (vld_ref[b] == 1))
    def _():
        pltpu.make_async_copy(W1_hbm.at[be_ref[b]], w1buf.at[bix_ref[b]],
                              sems.at[bix_ref[b]]).wait()
        pltpu.make_async_copy(W2_hbm.at[be_ref[b]], w2buf.at[bix_ref[b]],
                              sems.at[bix_ref[b]]).wait()

    @pl.when(vld_ref[b] == 1)
    def _():
        bix = bix_ref[b]
        x = x_ref[...].astype(jnp.bfloat16)
        h = jnp.maximum(jnp.dot(x, w1buf[bix].astype(jnp.bfloat16),
                                preferred_element_type=jnp.float32) + b1_ref[0], 0.0)
        y = jnp.dot(h.astype(jnp.bfloat16), w2buf[bix].astype(jnp.bfloat16),
                    preferred_element_type=jnp.float32) + b2_ref[0]
        y_ref[...] = y * w_ref[...]


def _ffn(x_g, wslot, block_expert, chg, bix, vld, W1, b1, W2, b2):
    grid_spec = pltpu.PrefetchScalarGridSpec(
        num_scalar_prefetch=4,
        grid=(_NB,),
        in_specs=[
            pl.BlockSpec((_BG, _D), lambda b, be, chg, bix, vld: (b, 0)),
            pl.BlockSpec((_BG, 1), lambda b, be, chg, bix, vld: (b, 0)),
            pl.BlockSpec((1, 1, _DFF), lambda b, be, chg, bix, vld: (be[b], 0, 0)),
            pl.BlockSpec((1, 1, _D), lambda b, be, chg, bix, vld: (be[b], 0, 0)),
            pl.BlockSpec(memory_space=pl.ANY),
            pl.BlockSpec(memory_space=pl.ANY),
        ],
        out_specs=pl.BlockSpec((_BG, _D), lambda b, be, chg, bix, vld: (b, 0)),
        scratch_shapes=[
            pltpu.VMEM((2, _D, _DFF), jnp.float32),
            pltpu.VMEM((2, _DFF, _D), jnp.float32),
            pltpu.SemaphoreType.DMA((2,)),
        ],
    )
    return pl.pallas_call(
        _ffn_body,
        grid_spec=grid_spec,
        out_shape=jax.ShapeDtypeStruct((_PMAX, _D), jnp.float32),
        compiler_params=pltpu.CompilerParams(
            dimension_semantics=("arbitrary",),
        ),
    )(block_expert, chg, bix, vld, x_g, wslot.reshape(_PMAX, 1),
      b1.reshape(_E, 1, _DFF), b2.reshape(_E, 1, _D), W1, W2)


_NW = 32
_TPW = _NTOK // _NW
_L = 16
_MESH = plsc.VectorSubcoreMesh(core_axis_name="c", subcore_axis_name="s")


def _wid():
    return lax.axis_index("s") * 2 + lax.axis_index("c")


@functools.partial(
    pl.kernel, mesh=_MESH,
    out_type=[
        jax.ShapeDtypeStruct((_PMAX, _D), jnp.float32),
        jax.ShapeDtypeStruct((_PMAX,), jnp.float32),
    ],
    scratch_types=[
        pltpu.VMEM((_TPW,), jnp.int32),
        pltpu.VMEM((_TPW,), jnp.int32),
        pltpu.VMEM((_TPW,), jnp.float32),
        pltpu.VMEM((_TPW,), jnp.float32),
        pltpu.VMEM((_TPW, _D), jnp.float32),
        pltpu.SemaphoreType.DMA,
        pltpu.SemaphoreType.DMA,
        pltpu.SemaphoreType.DMA,
    ],
)
def _sc_dispatch(x_hbm, pos0_hbm, pos1_hbm, w0_hbm, w1_hbm,
                 xg_hbm, wslot_hbm,
                 p0_v, p1_v, w0_v, w1_v, rows_v, sem0, sem1, semw):
    base = _wid() * _TPW
    pltpu.sync_copy(pos0_hbm.at[pl.ds(base, _TPW)], p0_v)
    pltpu.sync_copy(pos1_hbm.at[pl.ds(base, _TPW)], p1_v)
    pltpu.sync_copy(w0_hbm.at[pl.ds(base, _TPW)], w0_v)
    pltpu.sync_copy(w1_hbm.at[pl.ds(base, _TPW)], w1_v)
    pltpu.sync_copy(x_hbm.at[pl.ds(base, _TPW)], rows_v)
    c0 = pltpu.async_copy(rows_v, xg_hbm.at[p0_v], sem0)
    c1 = pltpu.async_copy(rows_v, xg_hbm.at[p1_v], sem1)
    cw0 = pltpu.async_copy(w0_v, wslot_hbm.at[p0_v], semw)
    cw1 = pltpu.async_copy(w1_v, wslot_hbm.at[p1_v], semw)
    c0.wait()
    c1.wait()
    cw0.wait()
    cw1.wait()


@functools.partial(
    pl.kernel, mesh=_MESH,
    out_type=jax.ShapeDtypeStruct((_NTOK, _D), jnp.float32),
    scratch_types=[
        pltpu.VMEM((_TPW,), jnp.int32),
        pltpu.VMEM((_TPW,), jnp.int32),
        pltpu.VMEM((_TPW, _D), jnp.float32),
        pltpu.VMEM((_TPW, _D), jnp.float32),
        pltpu.SemaphoreType.DMA,
        pltpu.SemaphoreType.DMA,
    ],
)
def _sc_combine(y_hbm, pos0_hbm, pos1_hbm, out_hbm,
                p0_v, p1_v, y0_v, y1_v, sem0, sem1):
    base = _wid() * _TPW
    pltpu.sync_copy(pos0_hbm.at[pl.ds(base, _TPW)], p0_v)
    pltpu.sync_copy(pos1_hbm.at[pl.ds(base, _TPW)], p1_v)
    c0 = pltpu.async_copy(y_hbm.at[p0_v], y0_v, sem0)
    c1 = pltpu.async_copy(y_hbm.at[p1_v], y1_v, sem1)
    c0.wait()
    c1.wait()

    def row(r, _):
        for c in range(_D // _L):
            sl = pl.ds(c * _L, _L)
            y0_v[r, sl] = y0_v[r, sl] + y1_v[r, sl]
        return 0

    lax.fori_loop(0, _TPW, row, 0)
    pltpu.sync_copy(y0_v, out_hbm.at[pl.ds(base, _TPW)])


def kernel(x, gate_w, gate_b, W1, b1, W2, b2):
    wts, idx = _gating(x, gate_w, gate_b)
    pos, binfo = _plan(idx)
    block_expert = binfo[:, 0]
    chg = binfo[:, 1]
    bix = binfo[:, 2]
    p0 = pos[:, 0]
    p1 = pos[:, 1]
    vld = binfo[:, 3]
    x_g, wslot = _sc_dispatch(x, p0, p1, wts[:, 0], wts[:, 1])
    y = _ffn(x_g, wslot, block_expert, chg, bix, vld, W1, b1, W2, b2)
    return _sc_combine(y, p0, p1)
